# Initial kernel scaffold; baseline (speedup 1.0000x reference)
import jax
import jax.numpy as jnp
from jax.experimental import pallas as pl


def kernel(input_attenuation):
    raise NotImplementedError("write your pallas kernel here")


# speedup vs baseline: 4.6540x; 4.6540x over previous
import jax
import jax.numpy as jnp
from jax.experimental import pallas as pl
from jax.experimental.pallas import tpu as pltpu

_N = 32
_TH = 1.1
_PAD_BEGIN = (_N - 1) // 2
_PAD_END = _N - 1 - _PAD_BEGIN


def _window_sum(v):
    T = v.shape[-1]
    a = v
    for k in (1, 2, 4, 8, 16):
        a = a + pltpu.roll(a, T - k, axis=1)
    return a


def _body(x_ref, out_ref, sig_ref):
    x = x_ref[...]
    T = x.shape[-1]

    s1 = _window_sum(x)
    s2 = _window_sum(x * x)

    s1 = pltpu.roll(s1, _PAD_BEGIN, axis=1)
    s2 = pltpu.roll(s2, _PAD_BEGIN, axis=1)

    inv_n = 1.0 / _N
    mean = s1 * inv_n
    var = jnp.maximum(s2 * inv_n - mean * mean, 0.0)
    sigma = jnp.sqrt(var)

    t_idx = jax.lax.broadcasted_iota(jnp.int32, x.shape, 1)
    valid = (t_idx >= _PAD_BEGIN) & (t_idx < T - _PAD_END)
    sigma = jnp.where(valid, sigma, 0.0)

    sigma_n = sigma * (1.0 / (2.0 * _TH))
    hard = jnp.clip(jnp.round(sigma_n), 0.0, 1.0)

    sig_ref[...] = sigma
    out_ref[...] = sigma_n + (hard - sigma_n)


@jax.jit
def kernel(input_attenuation):
    x = input_attenuation
    B, T = x.shape
    BR = 32
    grid = (B // BR,)
    spec = pl.BlockSpec((BR, T), lambda i: (i, 0))
    out, sig = pl.pallas_call(
        _body,
        grid=grid,
        in_specs=[spec],
        out_specs=[spec, spec],
        out_shape=[jax.ShapeDtypeStruct((B, T), x.dtype)] * 2,
        compiler_params=pltpu.CompilerParams(
            dimension_semantics=("parallel",),
            vmem_limit_bytes=100 * 1024 * 1024,
        ),
        name="stdwet_dry",
    )(x)
    return (out, sig)


# speedup vs baseline: 4.7855x; 1.0283x over previous
import jax
import jax.numpy as jnp
from jax.experimental import pallas as pl
from jax.experimental.pallas import tpu as pltpu

_N = 32
_TH = 1.1
_PAD_BEGIN = (_N - 1) // 2
_PAD_END = _N - 1 - _PAD_BEGIN
_CW = 2048
_HALO = 128


def _window_sum(v):
    w = v.shape[-1]
    a = v
    for k in (1, 2, 4, 8, 16):
        a = a + pltpu.roll(a, w - k, axis=1)
    return a


def _body(x_ref, out_ref, sig_ref):
    T = x_ref.shape[-1]
    n_chunks = T // _CW
    inv_n = 1.0 / _N
    scale = 1.0 / (2.0 * _TH)

    for c in range(n_chunks):
        c0 = c * _CW
        lo = max(c0 - _HALO, 0)
        hi = min(c0 + _CW + _HALO, T)
        base = c0 - lo
        width = hi - lo

        xs = x_ref[:, lo:hi]
        s1 = _window_sum(xs)
        s2 = _window_sum(xs * xs)

        mean = s1 * inv_n
        var = jnp.maximum(s2 * inv_n - mean * mean, 0.0)
        sigma_w = jnp.sqrt(var)
        sigma_b = pltpu.roll(sigma_w, _PAD_BEGIN, axis=1)[:, base:base + _CW]

        if c == 0 or c == n_chunks - 1:
            j = jax.lax.broadcasted_iota(jnp.int32, sigma_b.shape, 1)
            t_glob = j + c0
            valid = (t_glob >= _PAD_BEGIN) & (t_glob < T - _PAD_END)
            sigma_b = jnp.where(valid, sigma_b, 0.0)

        sigma_n = sigma_b * scale
        hard = jnp.clip(jnp.round(sigma_n), 0.0, 1.0)

        sig_ref[:, c0:c0 + _CW] = sigma_b
        out_ref[:, c0:c0 + _CW] = sigma_n + (hard - sigma_n)


@jax.jit
def kernel(input_attenuation):
    x = input_attenuation
    B, T = x.shape
    BR = 32
    grid = (B // BR,)
    spec = pl.BlockSpec((BR, T), lambda i: (i, 0))
    out, sig = pl.pallas_call(
        _body,
        grid=grid,
        in_specs=[spec],
        out_specs=[spec, spec],
        out_shape=[jax.ShapeDtypeStruct((B, T), x.dtype)] * 2,
        compiler_params=pltpu.CompilerParams(
            dimension_semantics=("parallel",),
            vmem_limit_bytes=100 * 1024 * 1024,
        ),
        name="stdwet_dry",
    )(x)
    return (out, sig)
